# baseline (speedup 1.0000x reference)
import functools

import jax
import jax.numpy as jnp
from jax import lax
from jax.experimental import pallas as pl
from jax.experimental.pallas import tpu as pltpu
from jax.experimental.pallas import tpu_sc as plsc

_NC = 2
_NS = 16
_NW = _NC * _NS
_L = 16

_TOTAL = 4096 * 128
_CHUNK = _TOTAL // _NW
_NIDX = 32
_NVALID = 26


def _body(f_hbm, idx_hbm, tv_hbm, w_hbm, out_hbm,
          idx_v, vals_v, w_v, fv, sem_f, sem_g):
    wid = lax.axis_index("s") * _NC + lax.axis_index("c")
    base = wid * _CHUNK

    cp_in = pltpu.make_async_copy(f_hbm.at[pl.ds(base, _CHUNK)], fv, sem_f)
    cp_in.start()

    pltpu.sync_copy(idx_hbm, idx_v)
    pltpu.sync_copy(w_hbm, w_v.at[pl.ds(0, 1)])
    pltpu.async_copy(tv_hbm.at[idx_v], vals_v, sem_g).wait()

    lane = lax.iota(jnp.int32, _L)
    dnums = lax.GatherDimensionNumbers(
        offset_dims=(), collapsed_slice_dims=(0,), start_index_map=(0,))

    def bcast_sum(x):
        for shift in (8, 4, 2, 1):
            x = x + lax.gather(
                x, (lane ^ shift)[:, None], dimension_numbers=dnums,
                slice_sizes=(1,), mode=lax.GatherScatterMode.PROMISE_IN_BOUNDS)
        return x

    v0 = vals_v[pl.ds(0, _L)]
    v1 = jnp.where(lane < (_NVALID - _L), vals_v[pl.ds(_L, _L)], 0.0)
    s16 = bcast_sum(v0 + v1)
    w16 = bcast_sum(jnp.where(lane == 0, w_v[...], 0.0))
    m = w16 * s16

    cp_in.wait()

    def step(i, carry):
        sl = pl.ds(i * _L, _L)
        fv[sl] = fv[sl] * m
        return carry

    lax.fori_loop(0, _CHUNK // _L, step, 0)
    pltpu.sync_copy(fv, out_hbm.at[pl.ds(base, _CHUNK)])


@jax.jit
def kernel(f, ti, tv, weight):
    idx = jnp.concatenate(
        [ti.astype(jnp.int32), jnp.zeros((_NIDX - _NVALID,), jnp.int32)])
    call = pl.kernel(
        _body,
        mesh=plsc.VectorSubcoreMesh(core_axis_name="c", subcore_axis_name="s"),
        out_type=jax.ShapeDtypeStruct((_TOTAL,), jnp.float32),
        scratch_types=[
            pltpu.VMEM((_NIDX,), jnp.int32),
            pltpu.VMEM((_NIDX,), jnp.float32),
            pltpu.VMEM((_L,), jnp.float32),
            pltpu.VMEM((_CHUNK,), jnp.float32),
            pltpu.SemaphoreType.DMA,
            pltpu.SemaphoreType.DMA,
        ],
    )
    out = call(f.reshape(_TOTAL), idx, tv, weight)
    return out.reshape(4096, 128)


# speedup vs baseline: 1.1905x; 1.1905x over previous
import functools

import jax
import jax.numpy as jnp
from jax import lax
from jax.experimental import pallas as pl
from jax.experimental.pallas import tpu as pltpu
from jax.experimental.pallas import tpu_sc as plsc

_NC = 2
_NS = 16
_NW = _NC * _NS
_L = 16

_TOTAL = 4096 * 128
_CHUNK = _TOTAL // _NW
_NIDX = 32
_NVALID = 26


def _body(f_hbm, idx_hbm, tv_hbm, w_hbm, out_hbm,
          idx_v, vals_v, w_v, fv, sem_f, sem_g):
    wid = lax.axis_index("s") * _NC + lax.axis_index("c")
    base = wid * _CHUNK

    cp_in = pltpu.make_async_copy(f_hbm.at[pl.ds(base, _CHUNK)], fv, sem_f)
    cp_in.start()

    pltpu.sync_copy(idx_hbm, idx_v)
    pltpu.sync_copy(w_hbm, w_v.at[pl.ds(0, 1)])
    pltpu.async_copy(tv_hbm.at[idx_v], vals_v, sem_g).wait()

    lane = lax.iota(jnp.int32, _L)
    dnums = lax.GatherDimensionNumbers(
        offset_dims=(), collapsed_slice_dims=(0,), start_index_map=(0,))

    def bcast_sum(x):
        for shift in (8, 4, 2, 1):
            x = x + lax.gather(
                x, (lane ^ shift)[:, None], dimension_numbers=dnums,
                slice_sizes=(1,), mode=lax.GatherScatterMode.PROMISE_IN_BOUNDS)
        return x

    v0 = vals_v[pl.ds(0, _L)]
    v1 = jnp.where(lane < (_NVALID - _L), vals_v[pl.ds(_L, _L)], 0.0)
    s16 = bcast_sum(v0 + v1)
    w16 = bcast_sum(jnp.where(lane == 0, w_v[...], 0.0))
    m = w16 * s16

    cp_in.wait()

    @plsc.parallel_loop(0, _CHUNK, step=_L, unroll=8)
    def _scale(i):
        fv[pl.ds(i, _L)] = fv[pl.ds(i, _L)] * m

    pltpu.sync_copy(fv, out_hbm.at[pl.ds(base, _CHUNK)])


@jax.jit
def kernel(f, ti, tv, weight):
    idx = jnp.concatenate(
        [ti.astype(jnp.int32), jnp.zeros((_NIDX - _NVALID,), jnp.int32)])
    call = pl.kernel(
        _body,
        mesh=plsc.VectorSubcoreMesh(core_axis_name="c", subcore_axis_name="s"),
        out_type=jax.ShapeDtypeStruct((_TOTAL,), jnp.float32),
        scratch_types=[
            pltpu.VMEM((_NIDX,), jnp.int32),
            pltpu.VMEM((_NIDX,), jnp.float32),
            pltpu.VMEM((_L,), jnp.float32),
            pltpu.VMEM((_CHUNK,), jnp.float32),
            pltpu.SemaphoreType.DMA,
            pltpu.SemaphoreType.DMA,
        ],
    )
    out = call(f.reshape(_TOTAL), idx, tv, weight)
    return out.reshape(4096, 128)
